# baseline (speedup 1.0000x reference)
import jax
import jax.numpy as jnp
from jax import lax
from jax.experimental import pallas as pl
from jax.experimental.pallas import tpu as pltpu

_NUM_CODES = 8192
_DIM = 32
_ROW_BLOCK = 256


def _gather_body(idx_ref, cb_ref, q_ref):
    idx = idx_ref[...]
    iota = lax.broadcasted_iota(jnp.int32, (idx.shape[0], _NUM_CODES), 1)
    onehot = jnp.where(iota == idx, 1.0, 0.0)
    q_ref[...] = lax.dot_general(
        onehot, cb_ref[...], (((1,), (0,)), ((), ())),
        preferred_element_type=jnp.float32,
        precision=lax.Precision.HIGHEST,
    )


def _gather_call(idx2, codebook):
    n_rows = idx2.shape[0]
    nb = n_rows // _ROW_BLOCK
    return pl.pallas_call(
        _gather_body,
        grid=(nb,),
        in_specs=[
            pl.BlockSpec((_ROW_BLOCK, 1), lambda i: (i, 0)),
            pl.BlockSpec((_NUM_CODES, _DIM), lambda i: (0, 0)),
        ],
        out_specs=pl.BlockSpec((_ROW_BLOCK, _DIM), lambda i: (i, 0)),
        out_shape=jax.ShapeDtypeStruct((n_rows, _DIM), jnp.float32),
    )(idx2, codebook)


def kernel(z, codebook):
    B, C, H, W = z.shape
    z_flattened = jnp.transpose(z, (0, 2, 3, 1))
    flat_z = z_flattened.reshape(-1, C)
    distances = (
        jnp.sum(flat_z ** 2, axis=1, keepdims=True)
        - 2.0 * jnp.matmul(flat_z, codebook.T)
        + jnp.sum(codebook ** 2, axis=1)
    )
    encoding_indices = jnp.argmin(distances, axis=1)

    qflat = _gather_call(encoding_indices[:, None], codebook)

    quantized = qflat.reshape(z_flattened.shape)
    quantized = jnp.transpose(quantized, (0, 3, 1, 2))
    e_latent_loss = jnp.mean((jax.lax.stop_gradient(quantized) - z) ** 2)
    q_latent_loss = jnp.mean((quantized - jax.lax.stop_gradient(z)) ** 2)
    loss = q_latent_loss + 0.25 * e_latent_loss
    quantized_st = z + jax.lax.stop_gradient(quantized - z)
    return (quantized_st, loss)


# speedup vs baseline: 2.2466x; 2.2466x over previous
import functools

import jax
import jax.numpy as jnp
from jax import lax
from jax.experimental import pallas as pl
from jax.experimental.pallas import tpu as pltpu
from jax.experimental.pallas import tpu_sc as plsc

_NUM_CODES = 8192
_DIM = 32
_IDX_CHUNK = 128


@functools.cache
def _codebook_gather(n_rows):
    info = plsc.get_sparse_core_info()
    n_cores, n_subcores = info.num_cores, info.num_subcores
    n_workers = n_cores * n_subcores
    b_per_w = n_rows // n_workers
    n_chunks = b_per_w // _IDX_CHUNK

    mesh = plsc.VectorSubcoreMesh(core_axis_name="c", subcore_axis_name="s")

    @functools.partial(
        pl.kernel,
        mesh=mesh,
        out_type=jax.ShapeDtypeStruct((n_rows, _DIM), jnp.float32),
        scratch_types=[
            pltpu.VMEM((n_chunks, _IDX_CHUNK), jnp.int32),
            pltpu.VMEM((n_chunks, _IDX_CHUNK, _DIM), jnp.float32),
            pltpu.SemaphoreType.DMA,
        ],
        compiler_params=pltpu.CompilerParams(use_tc_tiling_on_sc=False),
    )
    def gather(table_hbm, idx_hbm, out_hbm, idx_v, rows_v, sem):
        wid = lax.axis_index("s") * n_cores + lax.axis_index("c")
        base = wid * b_per_w
        copies = []
        for j in range(n_chunks):
            pltpu.sync_copy(idx_hbm.at[pl.ds(base + j * _IDX_CHUNK, _IDX_CHUNK)],
                            idx_v.at[j])
            copies.append(
                pltpu.async_copy(table_hbm.at[idx_v.at[j]], rows_v.at[j], sem))
        for j in range(n_chunks):
            copies[j].wait()
            pltpu.sync_copy(rows_v.at[j],
                            out_hbm.at[pl.ds(base + j * _IDX_CHUNK, _IDX_CHUNK)])

    return gather


def kernel(z, codebook):
    B, C, H, W = z.shape
    z_flattened = jnp.transpose(z, (0, 2, 3, 1))
    flat_z = z_flattened.reshape(-1, C)
    distances = (
        jnp.sum(flat_z ** 2, axis=1, keepdims=True)
        - 2.0 * jnp.matmul(flat_z, codebook.T)
        + jnp.sum(codebook ** 2, axis=1)
    )
    encoding_indices = jnp.argmin(distances, axis=1)

    qflat = _codebook_gather(flat_z.shape[0])(codebook, encoding_indices)

    quantized = qflat.reshape(z_flattened.shape)
    quantized = jnp.transpose(quantized, (0, 3, 1, 2))
    e_latent_loss = jnp.mean((jax.lax.stop_gradient(quantized) - z) ** 2)
    q_latent_loss = jnp.mean((quantized - jax.lax.stop_gradient(z)) ** 2)
    loss = q_latent_loss + 0.25 * e_latent_loss
    quantized_st = z + jax.lax.stop_gradient(quantized - z)
    return (quantized_st, loss)
